# baseline (speedup 1.0000x reference)
import jax
import jax.numpy as jnp
from jax.experimental import pallas as pl

HIDDEN = 2048
MEM = 100000
EMB = 128
BATCH = 1024
SEQ = 16

TILE = 4000
NTILES = MEM // TILE

_DN = (((1,), (1,)), ((), ()))


def _ln_gelu(y, g, beta):
    mu = jnp.mean(y, axis=1, keepdims=True)
    d = y - mu
    var = jnp.mean(d * d, axis=1, keepdims=True)
    z = d / jnp.sqrt(var + 1e-5) * g + beta
    return 0.5 * z * (1.0 + jax.lax.erf(z * (2.0 ** -0.5)))


def _encoder_body(x_ref, w1_ref, b1_ref, g1_ref, be1_ref,
                  w2_ref, b2_ref, g2_ref, be2_ref,
                  w3_ref, b3_ref, e_ref, en_ref):
    x = x_ref[...]
    y = jax.lax.dot_general(x, w1_ref[...], _DN,
                            preferred_element_type=jnp.float32) + b1_ref[...]
    y = _ln_gelu(y, g1_ref[...], be1_ref[...])
    y = jax.lax.dot_general(y, w2_ref[...], _DN,
                            preferred_element_type=jnp.float32) + b2_ref[...]
    y = _ln_gelu(y, g2_ref[...], be2_ref[...])
    e = jax.lax.dot_general(y, w3_ref[...], _DN,
                            preferred_element_type=jnp.float32) + b3_ref[...]
    n = jnp.sqrt(jnp.sum(e * e, axis=1, keepdims=True))
    e = e / jnp.maximum(n, 1e-12)
    e_ref[...] = e
    en_ref[...] = jnp.sqrt(jnp.sum(e * e, axis=1))[None, :]


def _retrieve_body(e_ref, en_ref, t_ref, val_ref, idx_ref):
    i = pl.program_id(0)
    t = t_ref[...]
    tn = jnp.sqrt(jnp.sum(t * t, axis=1, keepdims=True))
    num = jax.lax.dot_general(t, e_ref[...], _DN,
                              preferred_element_type=jnp.float32)
    scaled = num * (1.0 / jnp.maximum(tn, 1e-8))
    tmax = jnp.max(scaled, axis=0, keepdims=True)
    targ = (jnp.argmax(scaled, axis=0).astype(jnp.int32)
            + i * TILE)[None, :]

    @pl.when(i == 0)
    def _():
        val_ref[...] = tmax
        idx_ref[...] = targ

    @pl.when(i > 0)
    def _():
        prev = val_ref[...]
        better = tmax > prev
        val_ref[...] = jnp.where(better, tmax, prev)
        idx_ref[...] = jnp.where(better, targ, idx_ref[...])

    @pl.when(i == NTILES - 1)
    def _():
        val_ref[...] = val_ref[...] / jnp.maximum(en_ref[...], 1e-30)


def kernel(hidden_states, W1, b1, g1, beta1, W2, b2, g2, beta2, W3, b3,
           task_embeddings):
    hs2d = hidden_states.reshape(BATCH, SEQ * HIDDEN)
    row = lambda v: v.reshape(1, -1)
    h2, h4 = HIDDEN // 2, HIDDEN // 4
    full = lambda shape: pl.BlockSpec(shape, lambda i: (0, 0))

    e, en = pl.pallas_call(
        _encoder_body,
        grid=(1,),
        in_specs=[
            full((BATCH, HIDDEN)),
            full((h2, HIDDEN)), full((1, h2)), full((1, h2)), full((1, h2)),
            full((h4, h2)), full((1, h4)), full((1, h4)), full((1, h4)),
            full((EMB, h4)), full((1, EMB)),
        ],
        out_specs=(full((BATCH, EMB)), full((1, BATCH))),
        out_shape=(
            jax.ShapeDtypeStruct((BATCH, EMB), jnp.float32),
            jax.ShapeDtypeStruct((1, BATCH), jnp.float32),
        ),
    )(hs2d, W1, row(b1), row(g1), row(beta1),
      W2, row(b2), row(g2), row(beta2), W3, row(b3))

    val, idx = pl.pallas_call(
        _retrieve_body,
        grid=(NTILES,),
        in_specs=[
            full((BATCH, EMB)),
            full((1, BATCH)),
            pl.BlockSpec((TILE, EMB), lambda i: (i, 0)),
        ],
        out_specs=(full((1, BATCH)), full((1, BATCH))),
        out_shape=(
            jax.ShapeDtypeStruct((1, BATCH), jnp.float32),
            jax.ShapeDtypeStruct((1, BATCH), jnp.int32),
        ),
    )(e, en, task_embeddings)

    return val.reshape(BATCH), idx.reshape(BATCH)


# speedup vs baseline: 1.3885x; 1.3885x over previous
import jax
import jax.numpy as jnp
from jax.experimental import pallas as pl

HIDDEN = 2048
MEM = 100000
EMB = 128
BATCH = 1024
SEQ = 16

TILE = 4000
NTILES = MEM // TILE

_DN = (((1,), (1,)), ((), ()))


def _ln_gelu(y, g, beta):
    mu = jnp.mean(y, axis=1, keepdims=True)
    d = y - mu
    var = jnp.mean(d * d, axis=1, keepdims=True)
    z = d / jnp.sqrt(var + 1e-5) * g + beta
    return 0.5 * z * (1.0 + jax.lax.erf(z * (2.0 ** -0.5)))


def _encoder_body(x_ref, w1_ref, b1_ref, g1_ref, be1_ref,
                  w2_ref, b2_ref, g2_ref, be2_ref,
                  w3_ref, b3_ref, e_ref, en_ref):
    x = x_ref[...]
    y = jax.lax.dot_general(x, w1_ref[...], _DN,
                            preferred_element_type=jnp.float32) + b1_ref[...]
    y = _ln_gelu(y, g1_ref[...], be1_ref[...])
    y = jax.lax.dot_general(y, w2_ref[...], _DN,
                            preferred_element_type=jnp.float32) + b2_ref[...]
    y = _ln_gelu(y, g2_ref[...], be2_ref[...])
    e = jax.lax.dot_general(y, w3_ref[...], _DN,
                            preferred_element_type=jnp.float32) + b3_ref[...]
    n = jnp.sqrt(jnp.sum(e * e, axis=1, keepdims=True))
    e = e / jnp.maximum(n, 1e-12)
    e_ref[...] = e
    en_ref[...] = jnp.sqrt(jnp.sum(e * e, axis=1))[None, :]


def _retrieve_body(e_ref, en_ref, t_ref, val_ref, idx_ref):
    i = pl.program_id(0)
    t = t_ref[...]
    tn = jnp.sqrt(jnp.sum(t * t, axis=1, keepdims=True))
    num = jax.lax.dot_general(t, e_ref[...], _DN,
                              preferred_element_type=jnp.float32)
    scaled = num * (1.0 / jnp.maximum(tn, 1e-8))
    tmax = jnp.max(scaled, axis=0, keepdims=True)
    targ = (jnp.argmax(scaled, axis=0).astype(jnp.int32)
            + i * TILE)[None, :]

    @pl.when(i == 0)
    def _():
        val_ref[...] = tmax
        idx_ref[...] = targ

    @pl.when(i > 0)
    def _():
        prev = val_ref[...]
        better = tmax > prev
        val_ref[...] = jnp.where(better, tmax, prev)
        idx_ref[...] = jnp.where(better, targ, idx_ref[...])

    @pl.when(i == NTILES - 1)
    def _():
        val_ref[...] = val_ref[...] / jnp.maximum(en_ref[...], 1e-30)


def kernel(hidden_states, W1, b1, g1, beta1, W2, b2, g2, beta2, W3, b3,
           task_embeddings):
    x = hidden_states[:, 0]
    row = lambda v: v.reshape(1, -1)
    full = lambda shape: pl.BlockSpec(shape, lambda i: (0, 0))

    e, en = pl.pallas_call(
        _encoder_body,
        out_shape=(
            jax.ShapeDtypeStruct((BATCH, EMB), jnp.float32),
            jax.ShapeDtypeStruct((1, BATCH), jnp.float32),
        ),
    )(x, W1, row(b1), row(g1), row(beta1),
      W2, row(b2), row(g2), row(beta2), W3, row(b3))

    val, idx = pl.pallas_call(
        _retrieve_body,
        grid=(NTILES,),
        in_specs=[
            full((BATCH, EMB)),
            full((1, BATCH)),
            pl.BlockSpec((TILE, EMB), lambda i: (i, 0)),
        ],
        out_specs=(full((1, BATCH)), full((1, BATCH))),
        out_shape=(
            jax.ShapeDtypeStruct((1, BATCH), jnp.float32),
            jax.ShapeDtypeStruct((1, BATCH), jnp.int32),
        ),
    )(e, en, task_embeddings)

    return val.reshape(BATCH), idx.reshape(BATCH)


# speedup vs baseline: 1.5363x; 1.1064x over previous
import jax
import jax.numpy as jnp
from jax.experimental import pallas as pl

HIDDEN = 2048
MEM = 100000
EMB = 128
BATCH = 1024
SEQ = 16

TILE = 4000
NTILES = MEM // TILE

_DN = (((1,), (1,)), ((), ()))


def _ln_gelu(y, g, beta):
    mu = jnp.mean(y, axis=1, keepdims=True)
    d = y - mu
    var = jnp.mean(d * d, axis=1, keepdims=True)
    z = d / jnp.sqrt(var + 1e-5) * g + beta
    return 0.5 * z * (1.0 + jax.lax.erf(z * (2.0 ** -0.5)))


def _encoder_body(x_ref, w1_ref, b1_ref, g1_ref, be1_ref,
                  w2_ref, b2_ref, g2_ref, be2_ref,
                  w3_ref, b3_ref, e_ref, en_ref):
    x = x_ref[...]
    y = jax.lax.dot_general(x, w1_ref[...], _DN,
                            preferred_element_type=jnp.float32) + b1_ref[...]
    y = _ln_gelu(y, g1_ref[...], be1_ref[...])
    y = jax.lax.dot_general(y, w2_ref[...], _DN,
                            preferred_element_type=jnp.float32) + b2_ref[...]
    y = _ln_gelu(y, g2_ref[...], be2_ref[...])
    e = jax.lax.dot_general(y, w3_ref[...], _DN,
                            preferred_element_type=jnp.float32) + b3_ref[...]
    n = jnp.sqrt(jnp.sum(e * e, axis=1, keepdims=True))
    e = e / jnp.maximum(n, 1e-12)
    e_ref[...] = e
    en_ref[...] = jnp.sqrt(jnp.sum(e * e, axis=1))[None, :]


def _retrieve_body(e_ref, en_ref, t_ref, val_ref, idx_ref):
    i = pl.program_id(0)
    t = t_ref[...]
    tn = jnp.sqrt(jnp.sum(t * t, axis=1, keepdims=True))
    num = jax.lax.dot_general(t, e_ref[...], _DN,
                              preferred_element_type=jnp.float32)
    scaled = num * (1.0 / jnp.maximum(tn, 1e-8))
    s3 = scaled.reshape(TILE // 8, 8, BATCH)
    acc_v = s3[0]
    acc_i = jnp.zeros((8, BATCH), jnp.int32)
    for vr in range(1, TILE // 8):
        x = s3[vr]
        newer = x > acc_v
        acc_v = jnp.maximum(acc_v, x)
        acc_i = jnp.where(newer, vr, acc_i)
    tmax = jnp.max(acc_v, axis=0, keepdims=True)
    srow = jax.lax.broadcasted_iota(jnp.int32, (8, BATCH), 0)
    rows = jnp.where(acc_v == tmax, acc_i * 8 + srow, jnp.int32(TILE))
    targ = jnp.min(rows, axis=0, keepdims=True) + i * TILE

    @pl.when(i == 0)
    def _():
        val_ref[...] = tmax
        idx_ref[...] = targ

    @pl.when(i > 0)
    def _():
        prev = val_ref[...]
        better = tmax > prev
        val_ref[...] = jnp.where(better, tmax, prev)
        idx_ref[...] = jnp.where(better, targ, idx_ref[...])

    @pl.when(i == NTILES - 1)
    def _():
        val_ref[...] = val_ref[...] / jnp.maximum(en_ref[...], 1e-30)


def kernel(hidden_states, W1, b1, g1, beta1, W2, b2, g2, beta2, W3, b3,
           task_embeddings):
    x = hidden_states[:, 0]
    row = lambda v: v.reshape(1, -1)
    full = lambda shape: pl.BlockSpec(shape, lambda i: (0, 0))

    e, en = pl.pallas_call(
        _encoder_body,
        out_shape=(
            jax.ShapeDtypeStruct((BATCH, EMB), jnp.float32),
            jax.ShapeDtypeStruct((1, BATCH), jnp.float32),
        ),
    )(x, W1, row(b1), row(g1), row(beta1),
      W2, row(b2), row(g2), row(beta2), W3, row(b3))

    val, idx = pl.pallas_call(
        _retrieve_body,
        grid=(NTILES,),
        in_specs=[
            full((BATCH, EMB)),
            full((1, BATCH)),
            pl.BlockSpec((TILE, EMB), lambda i: (i, 0)),
        ],
        out_specs=(full((1, BATCH)), full((1, BATCH))),
        out_shape=(
            jax.ShapeDtypeStruct((1, BATCH), jnp.float32),
            jax.ShapeDtypeStruct((1, BATCH), jnp.int32),
        ),
    )(e, en, task_embeddings)

    return val.reshape(BATCH), idx.reshape(BATCH)


# speedup vs baseline: 1.5518x; 1.0101x over previous
import jax
import jax.numpy as jnp
from jax.experimental import pallas as pl

HIDDEN = 2048
MEM = 100000
EMB = 128
BATCH = 1024
SEQ = 16

TILE = 5000
NTILES = MEM // TILE

_DN = (((1,), (1,)), ((), ()))


def _ln_gelu(y, g, beta):
    mu = jnp.mean(y, axis=1, keepdims=True)
    d = y - mu
    var = jnp.mean(d * d, axis=1, keepdims=True)
    z = d / jnp.sqrt(var + 1e-5) * g + beta
    return 0.5 * z * (1.0 + jax.lax.erf(z * (2.0 ** -0.5)))


def _encoder_body(x_ref, w1_ref, b1_ref, g1_ref, be1_ref,
                  w2_ref, b2_ref, g2_ref, be2_ref,
                  w3_ref, b3_ref, e_ref, en_ref):
    x = x_ref[...]
    y = jax.lax.dot_general(x, w1_ref[...], _DN,
                            preferred_element_type=jnp.float32) + b1_ref[...]
    y = _ln_gelu(y, g1_ref[...], be1_ref[...])
    y = jax.lax.dot_general(y, w2_ref[...], _DN,
                            preferred_element_type=jnp.float32) + b2_ref[...]
    y = _ln_gelu(y, g2_ref[...], be2_ref[...])
    e = jax.lax.dot_general(y, w3_ref[...], _DN,
                            preferred_element_type=jnp.float32) + b3_ref[...]
    n = jnp.sqrt(jnp.sum(e * e, axis=1, keepdims=True))
    e = e / jnp.maximum(n, 1e-12)
    e_ref[...] = e
    en_ref[...] = jnp.sqrt(jnp.sum(e * e, axis=1))[None, :]


def _retrieve_body(e_ref, en_ref, t_ref, val_ref, idx_ref):
    i = pl.program_id(0)
    t = t_ref[...]
    tn = jnp.sqrt(jnp.sum(t * t, axis=1, keepdims=True))
    num = jax.lax.dot_general(t, e_ref[...], _DN,
                              preferred_element_type=jnp.float32)
    scaled = num * (1.0 / jnp.maximum(tn, 1e-8))
    s3 = scaled.reshape(TILE // 8, 8, BATCH)
    acc_v = s3[0]
    acc_i = jnp.zeros((8, BATCH), jnp.int32)
    for vr in range(1, TILE // 8):
        x = s3[vr]
        newer = x > acc_v
        acc_v = jnp.maximum(acc_v, x)
        acc_i = jnp.where(newer, vr, acc_i)
    tmax = jnp.max(acc_v, axis=0, keepdims=True)
    srow = jax.lax.broadcasted_iota(jnp.int32, (8, BATCH), 0)
    rows = jnp.where(acc_v == tmax, acc_i * 8 + srow, jnp.int32(TILE))
    targ = jnp.min(rows, axis=0, keepdims=True) + i * TILE

    @pl.when(i == 0)
    def _():
        val_ref[...] = tmax
        idx_ref[...] = targ

    @pl.when(i > 0)
    def _():
        prev = val_ref[...]
        better = tmax > prev
        val_ref[...] = jnp.where(better, tmax, prev)
        idx_ref[...] = jnp.where(better, targ, idx_ref[...])

    @pl.when(i == NTILES - 1)
    def _():
        val_ref[...] = val_ref[...] / jnp.maximum(en_ref[...], 1e-30)


def kernel(hidden_states, W1, b1, g1, beta1, W2, b2, g2, beta2, W3, b3,
           task_embeddings):
    x = hidden_states[:, 0]
    row = lambda v: v.reshape(1, -1)
    full = lambda shape: pl.BlockSpec(shape, lambda i: (0, 0))

    e, en = pl.pallas_call(
        _encoder_body,
        out_shape=(
            jax.ShapeDtypeStruct((BATCH, EMB), jnp.float32),
            jax.ShapeDtypeStruct((1, BATCH), jnp.float32),
        ),
    )(x, W1, row(b1), row(g1), row(beta1),
      W2, row(b2), row(g2), row(beta2), W3, row(b3))

    val, idx = pl.pallas_call(
        _retrieve_body,
        grid=(NTILES,),
        in_specs=[
            full((BATCH, EMB)),
            full((1, BATCH)),
            pl.BlockSpec((TILE, EMB), lambda i: (i, 0)),
        ],
        out_specs=(full((1, BATCH)), full((1, BATCH))),
        out_shape=(
            jax.ShapeDtypeStruct((1, BATCH), jnp.float32),
            jax.ShapeDtypeStruct((1, BATCH), jnp.int32),
        ),
    )(e, en, task_embeddings)

    return val.reshape(BATCH), idx.reshape(BATCH)


# speedup vs baseline: 1.5611x; 1.0060x over previous
import jax
import jax.numpy as jnp
from jax.experimental import pallas as pl
from jax.experimental.pallas import tpu as pltpu

HIDDEN = 2048
MEM = 100000
EMB = 128
BATCH = 1024
SEQ = 16

TILE = 5000
NTILES = MEM // TILE

_DN = (((1,), (1,)), ((), ()))


def _ln_gelu(y, g, beta):
    mu = jnp.mean(y, axis=1, keepdims=True)
    d = y - mu
    var = jnp.mean(d * d, axis=1, keepdims=True)
    z = d / jnp.sqrt(var + 1e-5) * g + beta
    return 0.5 * z * (1.0 + jax.lax.erf(z * (2.0 ** -0.5)))


def _fused_body(x_ref, w1_ref, b1_ref, g1_ref, be1_ref,
                w2_ref, b2_ref, g2_ref, be2_ref, w3_ref, b3_ref,
                t_ref, val_ref, idx_ref, e_scr, en_scr):
    i = pl.program_id(0)

    @pl.when(i == 0)
    def _():
        x = x_ref[...]
        y = jax.lax.dot_general(x, w1_ref[...], _DN,
                                preferred_element_type=jnp.float32) + b1_ref[...]
        y = _ln_gelu(y, g1_ref[...], be1_ref[...])
        y = jax.lax.dot_general(y, w2_ref[...], _DN,
                                preferred_element_type=jnp.float32) + b2_ref[...]
        y = _ln_gelu(y, g2_ref[...], be2_ref[...])
        e = jax.lax.dot_general(y, w3_ref[...], _DN,
                                preferred_element_type=jnp.float32) + b3_ref[...]
        n = jnp.sqrt(jnp.sum(e * e, axis=1, keepdims=True))
        e = e / jnp.maximum(n, 1e-12)
        e_scr[...] = e
        en_scr[...] = jnp.sqrt(jnp.sum(e * e, axis=1))[None, :]

    t = t_ref[...]
    tn = jnp.sqrt(jnp.sum(t * t, axis=1, keepdims=True))
    num = jax.lax.dot_general(t, e_scr[...], _DN,
                              preferred_element_type=jnp.float32)
    scaled = num * (1.0 / jnp.maximum(tn, 1e-8))
    s3 = scaled.reshape(TILE // 8, 8, BATCH)
    acc_v = s3[0]
    acc_i = jnp.zeros((8, BATCH), jnp.int32)
    for vr in range(1, TILE // 8):
        slab = s3[vr]
        newer = slab > acc_v
        acc_v = jnp.maximum(acc_v, slab)
        acc_i = jnp.where(newer, vr, acc_i)
    tmax = jnp.max(acc_v, axis=0, keepdims=True)
    srow = jax.lax.broadcasted_iota(jnp.int32, (8, BATCH), 0)
    rows = jnp.where(acc_v == tmax, acc_i * 8 + srow, jnp.int32(TILE))
    targ = jnp.min(rows, axis=0, keepdims=True) + i * TILE

    @pl.when(i == 0)
    def _():
        val_ref[...] = tmax
        idx_ref[...] = targ

    @pl.when(i > 0)
    def _():
        prev = val_ref[...]
        better = tmax > prev
        val_ref[...] = jnp.where(better, tmax, prev)
        idx_ref[...] = jnp.where(better, targ, idx_ref[...])

    @pl.when(i == NTILES - 1)
    def _():
        val_ref[...] = val_ref[...] / jnp.maximum(en_scr[...], 1e-30)


def kernel(hidden_states, W1, b1, g1, beta1, W2, b2, g2, beta2, W3, b3,
           task_embeddings):
    x = hidden_states[:, 0]
    row = lambda v: v.reshape(1, -1)
    h2, h4 = HIDDEN // 2, HIDDEN // 4
    const = lambda shape: pl.BlockSpec(shape, lambda i: (0, 0))

    val, idx = pl.pallas_call(
        _fused_body,
        grid=(NTILES,),
        in_specs=[
            const((BATCH, HIDDEN)),
            const((h2, HIDDEN)), const((1, h2)), const((1, h2)), const((1, h2)),
            const((h4, h2)), const((1, h4)), const((1, h4)), const((1, h4)),
            const((EMB, h4)), const((1, EMB)),
            pl.BlockSpec((TILE, EMB), lambda i: (i, 0)),
        ],
        out_specs=(const((1, BATCH)), const((1, BATCH))),
        out_shape=(
            jax.ShapeDtypeStruct((1, BATCH), jnp.float32),
            jax.ShapeDtypeStruct((1, BATCH), jnp.int32),
        ),
        scratch_shapes=[
            pltpu.VMEM((BATCH, EMB), jnp.float32),
            pltpu.VMEM((1, BATCH), jnp.float32),
        ],
    )(x, W1, row(b1), row(g1), row(beta1),
      W2, row(b2), row(g2), row(beta2), W3, row(b3), task_embeddings)

    return val.reshape(BATCH), idx.reshape(BATCH)
